# baseline (speedup 1.0000x reference)
import jax
import jax.numpy as jnp
from jax.experimental import pallas as pl
from jax.experimental.pallas import tpu as pltpu

_CHUNKS = 2


def _embed_kernel(src_hbm, out_ref, sems):
    rows = src_hbm.shape[0] // _CHUNKS
    copies = [
        pltpu.make_async_copy(
            src_hbm.at[pl.ds(i * rows, rows), :],
            out_ref.at[0, pl.ds(i * rows, rows), :],
            sems.at[i],
        )
        for i in range(_CHUNKS)
    ]
    for c in copies:
        c.start()
    for c in copies:
        c.wait()


def kernel(x, pos_table):
    maxlen = x.shape[-1]
    embed_dim = pos_table.shape[-1]
    return pl.pallas_call(
        _embed_kernel,
        in_specs=[pl.BlockSpec(memory_space=pl.ANY)],
        out_specs=pl.BlockSpec((1, maxlen, embed_dim), lambda: (0, 0, 0)),
        out_shape=jax.ShapeDtypeStruct((1, maxlen, embed_dim), pos_table.dtype),
        scratch_shapes=[pltpu.SemaphoreType.DMA((_CHUNKS,))],
    )(pos_table[:maxlen])


# speedup vs baseline: 1.0167x; 1.0167x over previous
import jax
import jax.numpy as jnp
from jax.experimental import pallas as pl
from jax.experimental.pallas import tpu as pltpu


def _embed_kernel(src_hbm, out_hbm, buf, sem):
    cin = pltpu.make_async_copy(src_hbm, buf, sem)
    cin.start()
    cin.wait()
    cout = pltpu.make_async_copy(buf, out_hbm.at[0], sem)
    cout.start()
    cout.wait()


def kernel(x, pos_table):
    maxlen = x.shape[-1]
    embed_dim = pos_table.shape[-1]
    return pl.pallas_call(
        _embed_kernel,
        in_specs=[pl.BlockSpec(memory_space=pl.ANY)],
        out_specs=pl.BlockSpec(memory_space=pl.ANY),
        out_shape=jax.ShapeDtypeStruct((1, maxlen, embed_dim), pos_table.dtype),
        scratch_shapes=[
            pltpu.VMEM((maxlen, embed_dim), pos_table.dtype),
            pltpu.SemaphoreType.DMA,
        ],
    )(pos_table[:maxlen])


# speedup vs baseline: 1.0173x; 1.0006x over previous
import jax
import jax.numpy as jnp
from jax.experimental import pallas as pl
from jax.experimental.pallas import tpu as pltpu


def _embed_kernel(src_hbm, out_hbm, buf, sem):
    cin = pltpu.make_async_copy(src_hbm, buf, sem)
    cin.start()
    cin.wait()
    cout = pltpu.make_async_copy(buf, out_hbm.at[0], sem)
    cout.start()
    cout.wait()


def kernel(x, pos_table):
    maxlen = x.shape[-1]
    embed_dim = pos_table.shape[-1]
    return pl.pallas_call(
        _embed_kernel,
        in_specs=[pl.BlockSpec(memory_space=pl.ANY)],
        out_specs=pl.BlockSpec(memory_space=pl.ANY),
        out_shape=jax.ShapeDtypeStruct((1, maxlen, embed_dim), pos_table.dtype),
        scratch_shapes=[
            pltpu.VMEM((maxlen, embed_dim), pos_table.dtype),
            pltpu.SemaphoreType.DMA,
        ],
        compiler_params=pltpu.CompilerParams(
            needs_layout_passes=False,
            internal_scratch_in_bytes=36864,
        ),
    )(pos_table[:maxlen])
